# baseline (speedup 1.0000x reference)
import functools

import jax
import jax.numpy as jnp
from jax import lax
from jax.experimental import pallas as pl
from jax.experimental.pallas import tpu as pltpu
from jax.experimental.pallas import tpu_sc as plsc

EMB = 64
SEQ = 50
N_IDX = 2 * SEQ
N_NONTEXT = 18
LANES = 16
NUM_CORES = 2
NUM_SUBCORES = 16
NUM_WORKERS = NUM_CORES * NUM_SUBCORES
NBUF = 2


def _tree_sum(vals):
    while len(vals) > 1:
        nxt = [vals[i] + vals[i + 1] for i in range(0, len(vals) - 1, 2)]
        if len(vals) % 2:
            nxt.append(vals[-1])
        vals = nxt
    return vals[0]


def _reduce_row(buf, out_v, r):
    mask_hi = jnp.int32(-65536)
    for h in range(2):
        for g2 in range(2):
            sl = pl.ds(g2 * 32, 32)
            lo, hi = [], []
            for i in range(SEQ):
                v = plsc.bitcast(buf[h * SEQ + i, sl], jnp.int32)
                lo.append(plsc.bitcast(v << 16, jnp.float32))
                hi.append(plsc.bitcast(v & mask_hi, jnp.float32))
            base = h * EMB + g2 * 32
            out_v[r, pl.ds(base, LANES)] = _tree_sum(lo)
            out_v[r, pl.ds(base + LANES, LANES)] = _tree_sum(hi)


def _make_pool(batch):
    b_per_w = batch // NUM_WORKERS
    chunk = min(b_per_w, 256)
    n_chunks = b_per_w // chunk

    @functools.partial(
        pl.kernel,
        out_type=jax.ShapeDtypeStruct((batch, 2 * EMB), jnp.float32),
        mesh=plsc.VectorSubcoreMesh(core_axis_name="c", subcore_axis_name="s"),
        scratch_types=[
            pltpu.VMEM((chunk, N_IDX), jnp.int32),
            [pltpu.VMEM((N_IDX, EMB), jnp.bfloat16) for _ in range(NBUF)],
            pltpu.VMEM((chunk, 2 * EMB), jnp.float32),
            [pltpu.SemaphoreType.DMA for _ in range(NBUF)],
        ],
        compiler_params=pltpu.CompilerParams(
            use_tc_tiling_on_sc=False, needs_layout_passes=False
        ),
    )
    def pool(idx_hbm, table_hbm, out_hbm, idx_v, rows, out_v, sems):
        wid = lax.axis_index("s") * NUM_CORES + lax.axis_index("c")

        def chunk_body(c, _):
            base = wid * b_per_w + c * chunk
            pltpu.sync_copy(idx_hbm.at[pl.ds(base, chunk)], idx_v)
            for p in range(NBUF - 1):
                pltpu.make_async_copy(
                    table_hbm.at[idx_v.at[p]], rows[p], sems[p]
                ).start()

            def group_body(jg, _):
                for b in range(NBUF):
                    r = NBUF * jg + b
                    nxt = r + NBUF - 1
                    nb = (b + NBUF - 1) % NBUF

                    @pl.when(nxt < chunk)
                    def _():
                        pltpu.make_async_copy(
                            table_hbm.at[idx_v.at[nxt]], rows[nb], sems[nb]
                        ).start()

                    pltpu.make_async_copy(
                        table_hbm.at[idx_v.at[r]], rows[b], sems[b]
                    ).wait()
                    _reduce_row(rows[b], out_v, r)
                return 0

            lax.fori_loop(0, chunk // NBUF, group_body, 0)
            pltpu.sync_copy(out_v, out_hbm.at[pl.ds(base, chunk)])
            return 0

        lax.fori_loop(0, n_chunks, chunk_body, 0)

    return pool


def _mlp_body(pooled_ref, xadd_ref, w1p_ref, w1a_ref, b1_ref, w2_ref, b2_ref,
              w3_ref, b3_ref, out_ref):
    nt = (((1,), (1,)), ((), ()))
    nn = (((1,), (0,)), ((), ()))
    f32 = jnp.float32
    h = lax.dot_general(w1p_ref[...], pooled_ref[...], nt, preferred_element_type=f32)
    h = h + lax.dot_general(w1a_ref[...], xadd_ref[...], nt, preferred_element_type=f32)
    h = jnp.maximum(h + b1_ref[...], 0.0)
    h = lax.dot_general(w2_ref[...], h, nn, preferred_element_type=f32)
    h = jnp.maximum(h + b2_ref[...], 0.0)
    h = lax.dot_general(w3_ref[...], h, nn, preferred_element_type=f32)
    out_ref[...] = jax.nn.sigmoid(h + b3_ref[...])


def _mlp(pooled, xadd, w1p, w1a, b1, w2, b2, w3, b3):
    batch = pooled.shape[0]
    blk = 2048
    full = lambda i: (0, 0)
    out = pl.pallas_call(
        _mlp_body,
        grid=(batch // blk,),
        in_specs=[
            pl.BlockSpec((blk, 2 * EMB), lambda i: (i, 0)),
            pl.BlockSpec((blk, N_NONTEXT), lambda i: (i, 0)),
            pl.BlockSpec((64, 2 * EMB), full),
            pl.BlockSpec((64, N_NONTEXT), full),
            pl.BlockSpec((64, 1), full),
            pl.BlockSpec((32, 64), full),
            pl.BlockSpec((32, 1), full),
            pl.BlockSpec((1, 32), full),
            pl.BlockSpec((1, 1), full),
        ],
        out_specs=pl.BlockSpec((1, blk), lambda i: (0, i)),
        out_shape=jax.ShapeDtypeStruct((1, batch), jnp.float32),
    )(pooled, xadd, w1p, w1a, b1, w2, b2, w3, b3)
    return out[0]


_POOL_PERM = [
    (p // 32) * 32 + (2 * (p % 32) if p % 32 < 16 else 2 * (p % 32 - 16) + 1)
    for p in range(4 * 32)
]


def kernel(x, emb_table, W1, b1, W2, b2, W3, b3):
    batch = x.shape[0]
    idx = x[:, :N_IDX].astype(jnp.int32)
    xadd = x[:, N_IDX:N_IDX + N_NONTEXT]
    pooled = _make_pool(batch)(idx, emb_table.astype(jnp.bfloat16))
    w1p = (W1[:, : 2 * EMB] * (1.0 / SEQ))[:, jnp.array(_POOL_PERM)]
    w1a = W1[:, 2 * EMB:]
    return _mlp(pooled, xadd, w1p, w1a, b1[:, None], W2, b2[:, None],
                W3, b3[:, None])


# speedup vs baseline: 1.4705x; 1.4705x over previous
import functools

import jax
import jax.numpy as jnp
from jax import lax
from jax.experimental import pallas as pl
from jax.experimental.pallas import tpu as pltpu
from jax.experimental.pallas import tpu_sc as plsc

EMB = 64
SEQ = 50
N_IDX = 2 * SEQ
N_NONTEXT = 18
LANES = 16
NUM_CORES = 2
NUM_SUBCORES = 16
NUM_WORKERS = NUM_CORES * NUM_SUBCORES
NBUF = 2


def _tree_sum(vals):
    while len(vals) > 1:
        nxt = [vals[i] + vals[i + 1] for i in range(0, len(vals) - 1, 2)]
        if len(vals) % 2:
            nxt.append(vals[-1])
        vals = nxt
    return vals[0]


def _reduce_row(buf, out_v, r):
    for h in range(2):
        for g in range(EMB // LANES):
            sl = pl.ds(g * LANES, LANES)
            vals = [buf[h * SEQ + i, sl] for i in range(SEQ)]
            out_v[r, pl.ds(h * EMB + g * LANES, LANES)] = _tree_sum(vals)


def _make_pool(batch):
    b_per_w = batch // NUM_WORKERS
    chunk = min(b_per_w, 256)
    n_chunks = b_per_w // chunk

    @functools.partial(
        pl.kernel,
        out_type=jax.ShapeDtypeStruct((batch, 2 * EMB), jnp.float32),
        mesh=plsc.VectorSubcoreMesh(core_axis_name="c", subcore_axis_name="s"),
        scratch_types=[
            pltpu.VMEM((chunk, N_IDX), jnp.int32),
            [pltpu.VMEM((N_IDX, EMB), jnp.float32) for _ in range(NBUF)],
            pltpu.VMEM((chunk, 2 * EMB), jnp.float32),
            [pltpu.SemaphoreType.DMA for _ in range(NBUF)],
        ],
        compiler_params=pltpu.CompilerParams(use_tc_tiling_on_sc=False),
    )
    def pool(idx_hbm, table_hbm, out_hbm, idx_v, rows, out_v, sems):
        wid = lax.axis_index("s") * NUM_CORES + lax.axis_index("c")

        def chunk_body(c, _):
            base = wid * b_per_w + c * chunk
            pltpu.sync_copy(idx_hbm.at[pl.ds(base, chunk)], idx_v)
            for p in range(NBUF - 1):
                pltpu.make_async_copy(
                    table_hbm.at[idx_v.at[p]], rows[p], sems[p]
                ).start()

            def group_body(jg, _):
                for b in range(NBUF):
                    r = NBUF * jg + b
                    nxt = r + NBUF - 1
                    nb = (b + NBUF - 1) % NBUF

                    @pl.when(nxt < chunk)
                    def _():
                        pltpu.make_async_copy(
                            table_hbm.at[idx_v.at[nxt]], rows[nb], sems[nb]
                        ).start()

                    pltpu.make_async_copy(
                        table_hbm.at[idx_v.at[r]], rows[b], sems[b]
                    ).wait()
                    _reduce_row(rows[b], out_v, r)
                return 0

            lax.fori_loop(0, chunk // NBUF, group_body, 0)
            pltpu.sync_copy(out_v, out_hbm.at[pl.ds(base, chunk)])
            return 0

        lax.fori_loop(0, n_chunks, chunk_body, 0)

    return pool


def _mlp_body(pooled_ref, xadd_ref, w1p_ref, w1a_ref, b1_ref, w2_ref, b2_ref,
              w3_ref, b3_ref, out_ref):
    nt = (((1,), (1,)), ((), ()))
    nn = (((1,), (0,)), ((), ()))
    f32 = jnp.float32
    h = lax.dot_general(w1p_ref[...], pooled_ref[...], nt, preferred_element_type=f32)
    h = h + lax.dot_general(w1a_ref[...], xadd_ref[...], nt, preferred_element_type=f32)
    h = jnp.maximum(h + b1_ref[...], 0.0)
    h = lax.dot_general(w2_ref[...], h, nn, preferred_element_type=f32)
    h = jnp.maximum(h + b2_ref[...], 0.0)
    h = lax.dot_general(w3_ref[...], h, nn, preferred_element_type=f32)
    out_ref[...] = jax.nn.sigmoid(h + b3_ref[...])


def _mlp(pooled, xadd, w1p, w1a, b1, w2, b2, w3, b3):
    batch = pooled.shape[0]
    blk = 2048
    full = lambda i: (0, 0)
    out = pl.pallas_call(
        _mlp_body,
        grid=(batch // blk,),
        in_specs=[
            pl.BlockSpec((blk, 2 * EMB), lambda i: (i, 0)),
            pl.BlockSpec((blk, N_NONTEXT), lambda i: (i, 0)),
            pl.BlockSpec((64, 2 * EMB), full),
            pl.BlockSpec((64, N_NONTEXT), full),
            pl.BlockSpec((64, 1), full),
            pl.BlockSpec((32, 64), full),
            pl.BlockSpec((32, 1), full),
            pl.BlockSpec((1, 32), full),
            pl.BlockSpec((1, 1), full),
        ],
        out_specs=pl.BlockSpec((1, blk), lambda i: (0, i)),
        out_shape=jax.ShapeDtypeStruct((1, batch), jnp.float32),
    )(pooled, xadd, w1p, w1a, b1, w2, b2, w3, b3)
    return out[0]


def kernel(x, emb_table, W1, b1, W2, b2, W3, b3):
    batch = x.shape[0]
    idx = x[:, :N_IDX].astype(jnp.int32)
    xadd = x[:, N_IDX:N_IDX + N_NONTEXT]
    tbl = emb_table + jnp.zeros((1, EMB), jnp.float32)
    pooled = _make_pool(batch)(idx, tbl)
    w1p = W1[:, : 2 * EMB] * (1.0 / SEQ)
    w1a = W1[:, 2 * EMB:]
    return _mlp(pooled, xadd, w1p, w1a, b1[:, None], W2, b2[:, None],
                W3, b3[:, None])


# speedup vs baseline: 1.7323x; 1.1781x over previous
import functools

import jax
import jax.numpy as jnp
from jax import lax
from jax.experimental import pallas as pl
from jax.experimental.pallas import tpu as pltpu
from jax.experimental.pallas import tpu_sc as plsc

EMB = 64
SEQ = 50
N_IDX = 2 * SEQ
N_NONTEXT = 18
LANES = 16
NUM_CORES = 2
NUM_SUBCORES = 16
NUM_WORKERS = NUM_CORES * NUM_SUBCORES
NBUF = 4


def _tree_sum(vals):
    while len(vals) > 1:
        nxt = [vals[i] + vals[i + 1] for i in range(0, len(vals) - 1, 2)]
        if len(vals) % 2:
            nxt.append(vals[-1])
        vals = nxt
    return vals[0]


def _reduce_row(buf, out_v, r):
    groups = [(h, g) for h in range(2) for g in range(EMB // LANES)]
    zero = jnp.zeros((LANES,), jnp.float32)

    def blk(i0, accs):
        out = []
        for a, (h, g) in zip(accs, groups):
            for i in range(10):
                a = a + buf[h * SEQ + i0 * 10 + i, pl.ds(g * LANES, LANES)]
            out.append(a)
        return tuple(out)

    accs = lax.fori_loop(0, SEQ // 10, blk, (zero,) * len(groups))
    for a, (h, g) in zip(accs, groups):
        out_v[r, pl.ds(h * EMB + g * LANES, LANES)] = a


def _make_pool(batch):
    b_per_w = batch // NUM_WORKERS
    chunk = min(b_per_w, 256)
    n_chunks = b_per_w // chunk

    @functools.partial(
        pl.kernel,
        out_type=jax.ShapeDtypeStruct((batch, 2 * EMB), jnp.float32),
        mesh=plsc.VectorSubcoreMesh(core_axis_name="c", subcore_axis_name="s"),
        scratch_types=[
            pltpu.VMEM((chunk, N_IDX), jnp.int32),
            [pltpu.VMEM((N_IDX, EMB), jnp.float32) for _ in range(NBUF)],
            pltpu.VMEM((chunk, 2 * EMB), jnp.float32),
            [pltpu.SemaphoreType.DMA for _ in range(NBUF)],
        ],
        compiler_params=pltpu.CompilerParams(use_tc_tiling_on_sc=False),
    )
    def pool(idx_hbm, table_hbm, out_hbm, idx_v, rows, out_v, sems):
        wid = lax.axis_index("s") * NUM_CORES + lax.axis_index("c")

        def chunk_body(c, _):
            base = wid * b_per_w + c * chunk
            pltpu.sync_copy(idx_hbm.at[pl.ds(base, chunk)], idx_v)
            for p in range(NBUF - 1):
                pltpu.make_async_copy(
                    table_hbm.at[idx_v.at[p]], rows[p], sems[p]
                ).start()

            def group_body(jg, _):
                for b in range(NBUF):
                    r = NBUF * jg + b
                    nxt = r + NBUF - 1
                    nb = (b + NBUF - 1) % NBUF

                    @pl.when(nxt < chunk)
                    def _():
                        pltpu.make_async_copy(
                            table_hbm.at[idx_v.at[nxt]], rows[nb], sems[nb]
                        ).start()

                    pltpu.make_async_copy(
                        table_hbm.at[idx_v.at[r]], rows[b], sems[b]
                    ).wait()
                    _reduce_row(rows[b], out_v, r)
                return 0

            lax.fori_loop(0, chunk // NBUF, group_body, 0)
            pltpu.sync_copy(out_v, out_hbm.at[pl.ds(base, chunk)])
            return 0

        lax.fori_loop(0, n_chunks, chunk_body, 0)

    return pool


def _mlp_body(pooled_ref, xadd_ref, w1p_ref, w1a_ref, b1_ref, w2_ref, b2_ref,
              w3_ref, b3_ref, out_ref):
    nt = (((1,), (1,)), ((), ()))
    nn = (((1,), (0,)), ((), ()))
    f32 = jnp.float32
    h = lax.dot_general(w1p_ref[...], pooled_ref[...], nt, preferred_element_type=f32)
    h = h + lax.dot_general(w1a_ref[...], xadd_ref[...], nt, preferred_element_type=f32)
    h = jnp.maximum(h + b1_ref[...], 0.0)
    h = lax.dot_general(w2_ref[...], h, nn, preferred_element_type=f32)
    h = jnp.maximum(h + b2_ref[...], 0.0)
    h = lax.dot_general(w3_ref[...], h, nn, preferred_element_type=f32)
    out_ref[...] = jax.nn.sigmoid(h + b3_ref[...])


def _mlp(pooled, xadd, w1p, w1a, b1, w2, b2, w3, b3):
    batch = pooled.shape[0]
    blk = 2048
    full = lambda i: (0, 0)
    out = pl.pallas_call(
        _mlp_body,
        grid=(batch // blk,),
        in_specs=[
            pl.BlockSpec((blk, 2 * EMB), lambda i: (i, 0)),
            pl.BlockSpec((blk, N_NONTEXT), lambda i: (i, 0)),
            pl.BlockSpec((64, 2 * EMB), full),
            pl.BlockSpec((64, N_NONTEXT), full),
            pl.BlockSpec((64, 1), full),
            pl.BlockSpec((32, 64), full),
            pl.BlockSpec((32, 1), full),
            pl.BlockSpec((1, 32), full),
            pl.BlockSpec((1, 1), full),
        ],
        out_specs=pl.BlockSpec((1, blk), lambda i: (0, i)),
        out_shape=jax.ShapeDtypeStruct((1, batch), jnp.float32),
    )(pooled, xadd, w1p, w1a, b1, w2, b2, w3, b3)
    return out[0]


def kernel(x, emb_table, W1, b1, W2, b2, W3, b3):
    batch = x.shape[0]
    idx = x[:, :N_IDX].astype(jnp.int32)
    xadd = x[:, N_IDX:N_IDX + N_NONTEXT]
    tbl = emb_table + jnp.zeros((1, EMB), jnp.float32)
    pooled = _make_pool(batch)(idx, tbl)
    w1p = W1[:, : 2 * EMB] * (1.0 / SEQ)
    w1a = W1[:, 2 * EMB:]
    return _mlp(pooled, xadd, w1p, w1a, b1[:, None], W2, b2[:, None],
                W3, b3[:, None])


# speedup vs baseline: 1.7899x; 1.0332x over previous
import functools

import jax
import jax.numpy as jnp
from jax import lax
from jax.experimental import pallas as pl
from jax.experimental.pallas import tpu as pltpu
from jax.experimental.pallas import tpu_sc as plsc

EMB = 64
SEQ = 50
N_IDX = 2 * SEQ
N_NONTEXT = 18
LANES = 16
NUM_CORES = 2
NUM_SUBCORES = 16
NUM_WORKERS = NUM_CORES * NUM_SUBCORES
NBUF = 8


def _tree_sum(vals):
    while len(vals) > 1:
        nxt = [vals[i] + vals[i + 1] for i in range(0, len(vals) - 1, 2)]
        if len(vals) % 2:
            nxt.append(vals[-1])
        vals = nxt
    return vals[0]


def _reduce_row(buf, out_v, r):
    groups = [(h, g) for h in range(2) for g in range(EMB // LANES)]
    zero = jnp.zeros((LANES,), jnp.float32)

    def blk(i0, accs):
        out = []
        for a, (h, g) in zip(accs, groups):
            for i in range(10):
                a = a + buf[h * SEQ + i0 * 10 + i, pl.ds(g * LANES, LANES)]
            out.append(a)
        return tuple(out)

    accs = lax.fori_loop(0, SEQ // 10, blk, (zero,) * len(groups))
    for a, (h, g) in zip(accs, groups):
        out_v[r, pl.ds(h * EMB + g * LANES, LANES)] = a


def _make_pool(batch):
    b_per_w = batch // NUM_WORKERS
    chunk = min(b_per_w, 256)
    n_chunks = b_per_w // chunk

    @functools.partial(
        pl.kernel,
        out_type=jax.ShapeDtypeStruct((batch, 2 * EMB), jnp.float32),
        mesh=plsc.VectorSubcoreMesh(core_axis_name="c", subcore_axis_name="s"),
        scratch_types=[
            pltpu.VMEM((chunk, N_IDX), jnp.int32),
            [pltpu.VMEM((N_IDX, EMB), jnp.float32) for _ in range(NBUF)],
            pltpu.VMEM((chunk, 2 * EMB), jnp.float32),
            [pltpu.SemaphoreType.DMA for _ in range(NBUF)],
        ],
        compiler_params=pltpu.CompilerParams(use_tc_tiling_on_sc=False),
    )
    def pool(idx_hbm, table_hbm, out_hbm, idx_v, rows, out_v, sems):
        wid = lax.axis_index("s") * NUM_CORES + lax.axis_index("c")

        def chunk_body(c, _):
            base = wid * b_per_w + c * chunk
            pltpu.sync_copy(idx_hbm.at[pl.ds(base, chunk)], idx_v)
            for p in range(NBUF - 1):
                pltpu.make_async_copy(
                    table_hbm.at[idx_v.at[p]], rows[p], sems[p]
                ).start()

            def group_body(jg, _):
                for b in range(NBUF):
                    r = NBUF * jg + b
                    nxt = r + NBUF - 1
                    nb = (b + NBUF - 1) % NBUF

                    @pl.when(nxt < chunk)
                    def _():
                        pltpu.make_async_copy(
                            table_hbm.at[idx_v.at[nxt]], rows[nb], sems[nb]
                        ).start()

                    pltpu.make_async_copy(
                        table_hbm.at[idx_v.at[r]], rows[b], sems[b]
                    ).wait()
                    _reduce_row(rows[b], out_v, r)
                return 0

            lax.fori_loop(0, chunk // NBUF, group_body, 0)
            pltpu.sync_copy(out_v, out_hbm.at[pl.ds(base, chunk)])
            return 0

        lax.fori_loop(0, n_chunks, chunk_body, 0)

    return pool


def _mlp_body(pooled_ref, xadd_ref, w1p_ref, w1a_ref, b1_ref, w2_ref, b2_ref,
              w3_ref, b3_ref, out_ref):
    nt = (((1,), (1,)), ((), ()))
    nn = (((1,), (0,)), ((), ()))
    f32 = jnp.float32
    h = lax.dot_general(w1p_ref[...], pooled_ref[...], nt, preferred_element_type=f32)
    h = h + lax.dot_general(w1a_ref[...], xadd_ref[...], nt, preferred_element_type=f32)
    h = jnp.maximum(h + b1_ref[...], 0.0)
    h = lax.dot_general(w2_ref[...], h, nn, preferred_element_type=f32)
    h = jnp.maximum(h + b2_ref[...], 0.0)
    h = lax.dot_general(w3_ref[...], h, nn, preferred_element_type=f32)
    out_ref[...] = jax.nn.sigmoid(h + b3_ref[...])


def _mlp(pooled, xadd, w1p, w1a, b1, w2, b2, w3, b3):
    batch = pooled.shape[0]
    blk = 2048
    full = lambda i: (0, 0)
    out = pl.pallas_call(
        _mlp_body,
        grid=(batch // blk,),
        in_specs=[
            pl.BlockSpec((blk, 2 * EMB), lambda i: (i, 0)),
            pl.BlockSpec((blk, N_NONTEXT), lambda i: (i, 0)),
            pl.BlockSpec((64, 2 * EMB), full),
            pl.BlockSpec((64, N_NONTEXT), full),
            pl.BlockSpec((64, 1), full),
            pl.BlockSpec((32, 64), full),
            pl.BlockSpec((32, 1), full),
            pl.BlockSpec((1, 32), full),
            pl.BlockSpec((1, 1), full),
        ],
        out_specs=pl.BlockSpec((1, blk), lambda i: (0, i)),
        out_shape=jax.ShapeDtypeStruct((1, batch), jnp.float32),
    )(pooled, xadd, w1p, w1a, b1, w2, b2, w3, b3)
    return out[0]


def kernel(x, emb_table, W1, b1, W2, b2, W3, b3):
    batch = x.shape[0]
    idx = x[:, :N_IDX].astype(jnp.int32)
    xadd = x[:, N_IDX:N_IDX + N_NONTEXT]
    tbl = emb_table + jnp.zeros((1, EMB), jnp.float32)
    pooled = _make_pool(batch)(idx, tbl)
    w1p = W1[:, : 2 * EMB] * (1.0 / SEQ)
    w1a = W1[:, 2 * EMB:]
    return _mlp(pooled, xadd, w1p, w1a, b1[:, None], W2, b2[:, None],
                W3, b3[:, None])
